# baseline (speedup 1.0000x reference)
import functools

import jax
import jax.numpy as jnp
from jax import lax
from jax.experimental import pallas as pl
from jax.experimental.pallas import tpu as pltpu
from jax.experimental.pallas import tpu_sc as plsc

F32 = jnp.float32
_NC = 2
_NS = 16
_LN = 16
_BB = 128


def _tc_sigmoid(v):
    return 0.5 * jnp.tanh(0.5 * v) + 0.5


def _lane_bcast(v, lane):
    idx = jnp.full((_LN, 1), lane, jnp.int32)
    dn = lax.GatherDimensionNumbers(offset_dims=(), collapsed_slice_dims=(0,),
                                    start_index_map=(0,))
    return lax.gather(v, idx, dn, (1,),
                      mode=lax.GatherScatterMode.PROMISE_IN_BOUNDS)


def _sc_mesh():
    return plsc.VectorSubcoreMesh(core_axis_name="c", subcore_axis_name="s",
                                  num_cores=_NC, num_subcores=_NS)


@functools.lru_cache(maxsize=None)
def _make_deg_kernel(npad, epd):
    chunk = epd // (_NC * _NS)
    nvec = chunk // _LN
    per_tile = npad // _NS

    def body(col_hbm, ew_hbm, out_hbm, colv, eww, acc, redbuf, shared):
        c = lax.axis_index("c")
        s = lax.axis_index("s")
        wid = c * _NS + s

        def zero(i, _):
            acc[pl.ds(i * _LN, _LN)] = jnp.zeros((_LN,), F32)
            return 0
        lax.fori_loop(0, npad // _LN, zero, 0)

        pltpu.sync_copy(col_hbm.at[wid], colv)
        pltpu.sync_copy(ew_hbm.at[wid], eww)

        def step(i, _):
            idx = colv[pl.ds(i * _LN, _LN)]
            w = eww[pl.ds(i * _LN, _LN)]
            plsc.addupdate_scatter(acc, [idx], w)
            return 0
        lax.fori_loop(0, nvec, step, 0)

        pltpu.sync_copy(acc, shared.at[s])
        plsc.subcore_barrier()

        base = s * per_tile
        for t in range(_NS):
            pltpu.sync_copy(shared.at[t, pl.ds(base, per_tile)], redbuf.at[t])

        def red(j, _):
            sl = pl.ds(j * _LN, _LN)
            v = redbuf[0, sl]
            for t in range(1, _NS):
                v = v + redbuf[t, sl]
            acc[sl] = v
            return 0
        lax.fori_loop(0, per_tile // _LN, red, 0)
        pltpu.sync_copy(acc.at[pl.ds(0, per_tile)],
                        out_hbm.at[c, pl.ds(base, per_tile)])

    return pl.kernel(
        body,
        out_type=jax.ShapeDtypeStruct((_NC, npad), F32),
        mesh=_sc_mesh(),
        scratch_types=[
            pltpu.VMEM((chunk,), jnp.int32),
            pltpu.VMEM((chunk,), F32),
            pltpu.VMEM((npad,), F32),
            pltpu.VMEM((_NS, per_tile), F32),
            pltpu.VMEM_SHARED((_NS, npad), F32),
        ],
        compiler_params=pltpu.CompilerParams(needs_layout_passes=False),
        name="sc_degree",
    )


_NBUF = 4


@functools.lru_cache(maxsize=None)
def _make_agg_kernel(n, d, nb, nrep):
    rows_per_tile = n // _NS
    nvec = d // _LN
    assert nb % _NBUF == 0 and nb >= 2 * _NBUF

    def body(*args):
        ys = args[0:2 * nrep]
        row_hbm, col_hbm, ew_hbm = args[2 * nrep:2 * nrep + 3]
        outs = args[2 * nrep + 3:4 * nrep + 3]
        (idxr, idxc, eww, rb0, rb1, rb2, rb3, acc,
         gs0, gs1, gs2, gs3, ss0, ss1, ss2, ss3) = args[4 * nrep + 3:]
        c = lax.axis_index("c")
        s = lax.axis_index("s")
        base = s * rows_per_tile
        bufs = (rb0, rb1, rb2, rb3)
        gsems = (gs0, gs1, gs2, gs3)
        ssems = (ss0, ss1, ss2, ss3)

        pltpu.sync_copy(row_hbm.at[s], idxr)
        pltpu.sync_copy(col_hbm.at[s], idxc)
        pltpu.sync_copy(ew_hbm.at[s], eww)

        def run(y_hbm, out_hbm):
            pltpu.sync_copy(y_hbm.at[pl.ds(base, rows_per_tile)],
                            acc.at[pl.ds(base, rows_per_tile)])
            plsc.subcore_barrier()

            for p in range(2):
                pltpu.async_copy(y_hbm.at[idxr.at[p]], bufs[p], gsems[p])

            def outer(j, _):
                j0 = j * _NBUF
                for b in range(_NBUF):
                    jj = j0 + b
                    p = b
                    q = (b + 2) % _NBUF
                    buf = bufs[p]
                    pltpu.make_async_copy(y_hbm.at[idxr.at[jj]], buf,
                                          gsems[p]).wait()

                    @pl.when(jj + 2 < nb)
                    def _():
                        @pl.when(jj >= 2)
                        def _():
                            pltpu.make_async_copy(
                                bufs[q], acc.at[idxc.at[jj - 2]],
                                ssems[q]).wait()
                        pltpu.async_copy(y_hbm.at[idxr.at[jj + 2]], bufs[q],
                                         gsems[q])

                    def grp(g, _):
                        wv = eww[jj, pl.ds(g * _LN, _LN)]
                        for l in range(_LN):
                            w = _lane_bcast(wv, l)
                            eb = g * _LN + l
                            for f in range(nvec):
                                sl = pl.ds(f * _LN, _LN)
                                buf[eb, sl] = buf[eb, sl] * w
                        return 0
                    lax.fori_loop(0, _BB // _LN, grp, 0)

                    pltpu.async_copy(buf, acc.at[idxc.at[jj]], ssems[p],
                                     add=True)
                return 0
            lax.fori_loop(0, nb // _NBUF, outer, 0)

            for p in range(_NBUF):
                pltpu.make_async_copy(bufs[p], acc.at[idxc.at[0]],
                                      ssems[p]).wait()

            plsc.subcore_barrier()
            pltpu.sync_copy(acc.at[pl.ds(base, rows_per_tile)],
                            out_hbm.at[pl.ds(base, rows_per_tile)])
            plsc.subcore_barrier()

        for rep in range(nrep):
            @pl.when(c == 0)
            def _():
                run(ys[2 * rep], outs[2 * rep])

            @pl.when(c == 1)
            def _():
                run(ys[2 * rep + 1], outs[2 * rep + 1])

    return pl.kernel(
        body,
        out_type=[jax.ShapeDtypeStruct((n, d), F32)] * (2 * nrep),
        mesh=_sc_mesh(),
        scratch_types=[
            pltpu.VMEM((nb, _BB), jnp.int32),
            pltpu.VMEM((nb, _BB), jnp.int32),
            pltpu.VMEM((nb, _BB), F32),
            pltpu.VMEM((_BB, d), F32),
            pltpu.VMEM((_BB, d), F32),
            pltpu.VMEM((_BB, d), F32),
            pltpu.VMEM((_BB, d), F32),
            pltpu.VMEM_SHARED((n, d), F32),
            pltpu.SemaphoreType.DMA,
            pltpu.SemaphoreType.DMA,
            pltpu.SemaphoreType.DMA,
            pltpu.SemaphoreType.DMA,
            pltpu.SemaphoreType.DMA,
            pltpu.SemaphoreType.DMA,
            pltpu.SemaphoreType.DMA,
            pltpu.SemaphoreType.DMA,
        ],
        compiler_params=pltpu.CompilerParams(use_tc_tiling_on_sc=False,
                                             needs_layout_passes=False),
        name=f"sc_edge_agg_{d}",
    )


def _t1_call(x, W1, degp):
    n, f = x.shape
    h = W1.shape[1]
    rb = 1000

    hq = h // 4

    def body(x_ref, w_ref, degp_ref, dinv_ref, y0_ref, y1_ref, y2_ref, y3_ref):
        deg = 1.0 + degp_ref[:, 0:1] + degp_ref[:, 1:2]
        dinv = lax.rsqrt(deg)
        dinv_ref[...] = dinv
        y = dinv * jnp.dot(x_ref[...], w_ref[...], preferred_element_type=F32)
        y0_ref[...] = y[:, 0 * hq:1 * hq]
        y1_ref[...] = y[:, 1 * hq:2 * hq]
        y2_ref[...] = y[:, 2 * hq:3 * hq]
        y3_ref[...] = y[:, 3 * hq:4 * hq]

    return pl.pallas_call(
        body,
        grid=(n // rb,),
        in_specs=[pl.BlockSpec((rb, f), lambda i: (i, 0)),
                  pl.BlockSpec((f, h), lambda i: (0, 0)),
                  pl.BlockSpec((rb, 2), lambda i: (i, 0))],
        out_specs=[pl.BlockSpec((rb, 1), lambda i: (i, 0))]
        + [pl.BlockSpec((rb, hq), lambda i: (i, 0))] * 4,
        out_shape=[jax.ShapeDtypeStruct((n, 1), F32)]
        + [jax.ShapeDtypeStruct((n, hq), F32)] * 4,
    )(x, W1, degp)


def _t2_call(aggs, dinv, b1r, Wc):
    n, hq = aggs[0].shape
    h = 4 * hq
    d2 = Wc.shape[1]
    rb = 1000

    def body(a0_ref, a1_ref, a2_ref, a3_ref, dinv_ref, b1_ref, wc_ref,
             ya_ref, yb_ref):
        dinv = dinv_ref[...]
        y2 = jnp.zeros((rb, d2), F32)
        for k, a_ref in enumerate((a0_ref, a1_ref, a2_ref, a3_ref)):
            x1k = _tc_sigmoid(dinv * a_ref[...]
                              + b1_ref[:, k * hq:(k + 1) * hq])
            y2 = y2 + jnp.dot(x1k, wc_ref[k * hq:(k + 1) * hq, :],
                              preferred_element_type=F32)
        y2 = dinv * y2
        ya_ref[...] = y2[:, : d2 // 2]
        yb_ref[...] = y2[:, d2 // 2:]

    return pl.pallas_call(
        body,
        grid=(n // rb,),
        in_specs=[pl.BlockSpec((rb, hq), lambda i: (i, 0))] * 4
        + [pl.BlockSpec((rb, 1), lambda i: (i, 0)),
           pl.BlockSpec((1, h), lambda i: (0, 0)),
           pl.BlockSpec((h, d2), lambda i: (0, 0))],
        out_specs=[pl.BlockSpec((rb, d2 // 2), lambda i: (i, 0)),
                   pl.BlockSpec((rb, d2 // 2), lambda i: (i, 0))],
        out_shape=[jax.ShapeDtypeStruct((n, d2 // 2), F32),
                   jax.ShapeDtypeStruct((n, d2 // 2), F32)],
    )(*aggs, dinv, b1r, Wc)


def _t3_call(agg2a, agg2b, dinv, b2r, b3r):
    n, o = agg2a.shape
    rb = 200

    def body(ai_ref, afull_ref, bi_ref, dvi_ref, dvfull_ref, b2_ref, b3_ref,
             adj_ref, mu_ref, lv_ref):
        mu_i = _tc_sigmoid(dvi_ref[...] * ai_ref[...] + b2_ref[...])
        lv_i = _tc_sigmoid(dvi_ref[...] * bi_ref[...] + b3_ref[...])
        mu_all = _tc_sigmoid(dvfull_ref[...] * afull_ref[...] + b2_ref[...])
        mu_ref[...] = mu_i
        lv_ref[...] = lv_i
        prod = lax.dot_general(mu_i, mu_all, (((1,), (1,)), ((), ())),
                               preferred_element_type=F32)
        adj_ref[...] = _tc_sigmoid(prod)

    return pl.pallas_call(
        body,
        grid=(n // rb,),
        in_specs=[pl.BlockSpec((rb, o), lambda i: (i, 0)),
                  pl.BlockSpec((n, o), lambda i: (0, 0)),
                  pl.BlockSpec((rb, o), lambda i: (i, 0)),
                  pl.BlockSpec((rb, 1), lambda i: (i, 0)),
                  pl.BlockSpec((n, 1), lambda i: (0, 0)),
                  pl.BlockSpec((1, o), lambda i: (0, 0)),
                  pl.BlockSpec((1, o), lambda i: (0, 0))],
        out_specs=[pl.BlockSpec((rb, n), lambda i: (i, 0)),
                   pl.BlockSpec((rb, o), lambda i: (i, 0)),
                   pl.BlockSpec((rb, o), lambda i: (i, 0))],
        out_shape=[jax.ShapeDtypeStruct((n, n), F32),
                   jax.ShapeDtypeStruct((n, o), F32),
                   jax.ShapeDtypeStruct((n, o), F32)],
    )(agg2a, agg2a, agg2b, dinv, dinv, b2r, b3r)


def kernel(x, edge_index, edge_weight, W1, b1, W2, b2, W3, b3):
    n, f = x.shape
    h = W1.shape[1]
    o = W2.shape[1]
    e = edge_weight.shape[0]

    row = edge_index[0].astype(jnp.int32)
    col = edge_index[1].astype(jnp.int32)
    ew = edge_weight.astype(F32)

    nb = -(-(-(-e // (_NS * _BB))) // _NBUF) * _NBUF
    ep = _NS * nb * _BB
    chunk = -(-e // (_NC * _NS * _LN)) * _LN
    epd = _NC * _NS * chunk
    epmax = max(ep, epd)
    padlen = epmax - e
    rowp = jnp.concatenate([row, jnp.zeros((padlen,), jnp.int32)])
    colp = jnp.concatenate([col, jnp.zeros((padlen,), jnp.int32)])
    ewp = jnp.concatenate([ew, jnp.zeros((padlen,), F32)])
    row_s = rowp[:ep].reshape(_NS, nb, _BB)
    col_s = colp[:ep].reshape(_NS, nb, _BB)
    ew_s = ewp[:ep].reshape(_NS, nb, _BB)
    col_d = colp[:epd].reshape(_NC * _NS, chunk)
    ew_d = ewp[:epd].reshape(_NC * _NS, chunk)

    npad = -(-n // (_NS * _LN)) * (_NS * _LN)

    degp = _make_deg_kernel(npad, epd)(col_d, ew_d)
    degp2 = degp[:, :n].T

    dinv, y1q0, y1q1, y1q2, y1q3 = _t1_call(x, W1, degp2)
    agg1 = _make_agg_kernel(n, h // 4, nb, 2)(
        y1q0, y1q1, y1q2, y1q3, row_s, col_s, ew_s)

    Wc = jnp.concatenate([W2, W3], axis=1)
    y2a, y2b = _t2_call(agg1, dinv, b1.reshape(1, h), Wc)
    agg2a, agg2b = _make_agg_kernel(n, o, nb, 1)(
        y2a, y2b, row_s, col_s, ew_s)

    adj, mu, logvar = _t3_call(agg2a, agg2b, dinv,
                               b2.reshape(1, o), b3.reshape(1, o))
    return mu, logvar, mu, adj


# speedup vs baseline: 1.0348x; 1.0348x over previous
import functools

import jax
import jax.numpy as jnp
from jax import lax
from jax.experimental import pallas as pl
from jax.experimental.pallas import tpu as pltpu
from jax.experimental.pallas import tpu_sc as plsc

F32 = jnp.float32
_NC = 2
_NS = 16
_LN = 16
_BB = 128


def _tc_sigmoid(v):
    return 0.5 * jnp.tanh(0.5 * v) + 0.5


def _lane_bcast(v, lane):
    idx = jnp.full((_LN, 1), lane, jnp.int32)
    dn = lax.GatherDimensionNumbers(offset_dims=(), collapsed_slice_dims=(0,),
                                    start_index_map=(0,))
    return lax.gather(v, idx, dn, (1,),
                      mode=lax.GatherScatterMode.PROMISE_IN_BOUNDS)


def _sc_mesh():
    return plsc.VectorSubcoreMesh(core_axis_name="c", subcore_axis_name="s",
                                  num_cores=_NC, num_subcores=_NS)


@functools.lru_cache(maxsize=None)
def _make_deg_kernel(npad, epd):
    chunk = epd // (_NC * _NS)
    nvec = chunk // _LN
    per_tile = npad // _NS

    def body(col_hbm, ew_hbm, out_hbm, colv, eww, acc, redbuf, shared):
        c = lax.axis_index("c")
        s = lax.axis_index("s")
        wid = c * _NS + s

        def zero(i, _):
            acc[pl.ds(i * _LN, _LN)] = jnp.zeros((_LN,), F32)
            return 0
        lax.fori_loop(0, npad // _LN, zero, 0)

        pltpu.sync_copy(col_hbm.at[wid], colv)
        pltpu.sync_copy(ew_hbm.at[wid], eww)

        def step(i, _):
            idx = colv[pl.ds(i * _LN, _LN)]
            w = eww[pl.ds(i * _LN, _LN)]
            plsc.addupdate_scatter(acc, [idx], w)
            return 0
        lax.fori_loop(0, nvec, step, 0)

        pltpu.sync_copy(acc, shared.at[s])
        plsc.subcore_barrier()

        base = s * per_tile
        for t in range(_NS):
            pltpu.sync_copy(shared.at[t, pl.ds(base, per_tile)], redbuf.at[t])

        def red(j, _):
            sl = pl.ds(j * _LN, _LN)
            v = redbuf[0, sl]
            for t in range(1, _NS):
                v = v + redbuf[t, sl]
            acc[sl] = v
            return 0
        lax.fori_loop(0, per_tile // _LN, red, 0)
        pltpu.sync_copy(acc.at[pl.ds(0, per_tile)],
                        out_hbm.at[c, pl.ds(base, per_tile)])

    return pl.kernel(
        body,
        out_type=jax.ShapeDtypeStruct((_NC, npad), F32),
        mesh=_sc_mesh(),
        scratch_types=[
            pltpu.VMEM((chunk,), jnp.int32),
            pltpu.VMEM((chunk,), F32),
            pltpu.VMEM((npad,), F32),
            pltpu.VMEM((_NS, per_tile), F32),
            pltpu.VMEM_SHARED((_NS, npad), F32),
        ],
        compiler_params=pltpu.CompilerParams(needs_layout_passes=False),
        name="sc_degree",
    )


_NBUF = 2


@functools.lru_cache(maxsize=None)
def _make_agg_kernel(n, d, nb, nrep):
    rows_per_tile = n // _NS
    nvec = d // _LN
    assert nb % _NBUF == 0 and nb >= 2 * _NBUF

    def body(*args):
        ys = args[0:2 * nrep]
        row_hbm, col_hbm, ew_hbm = args[2 * nrep:2 * nrep + 3]
        outs = args[2 * nrep + 3:4 * nrep + 3]
        (idxr, idxc, eww, rb0, rb1, ysp, acc,
         gs0, gs1, ss0, ss1) = args[4 * nrep + 3:]
        c = lax.axis_index("c")
        s = lax.axis_index("s")
        base = s * rows_per_tile
        rows = pl.ds(base, rows_per_tile)
        bufs = (rb0, rb1)
        gsems = (gs0, gs1)
        ssems = (ss0, ss1)

        pltpu.sync_copy(row_hbm.at[s], idxr)
        pltpu.sync_copy(col_hbm.at[s], idxc)
        pltpu.sync_copy(ew_hbm.at[s], eww)

        def run(y_hbm, out_hbm):
            pltpu.sync_copy(y_hbm.at[rows], ysp.at[rows])
            pltpu.sync_copy(y_hbm.at[rows], acc.at[rows])
            plsc.subcore_barrier()

            def outer(j, _):
                j0 = j * _NBUF
                for p in range(_NBUF):
                    jj = j0 + p
                    buf = bufs[p]

                    @pl.when(jj >= _NBUF)
                    def _():
                        pltpu.make_async_copy(
                            buf, acc.at[idxc.at[jj - _NBUF]],
                            ssems[p]).wait()

                    pltpu.async_copy(ysp.at[idxr.at[jj]], buf,
                                     gsems[p]).wait()

                    def grp(g, _):
                        wv = eww[jj, pl.ds(g * _LN, _LN)]
                        for l in range(_LN):
                            w = _lane_bcast(wv, l)
                            eb = g * _LN + l
                            for f in range(nvec):
                                sl = pl.ds(f * _LN, _LN)
                                buf[eb, sl] = buf[eb, sl] * w
                        return 0
                    lax.fori_loop(0, _BB // _LN, grp, 0)

                    pltpu.async_copy(buf, acc.at[idxc.at[jj]], ssems[p],
                                     add=True)
                return 0
            lax.fori_loop(0, nb // _NBUF, outer, 0)

            for p in range(_NBUF):
                pltpu.make_async_copy(bufs[p], acc.at[idxc.at[0]],
                                      ssems[p]).wait()

            plsc.subcore_barrier()
            pltpu.sync_copy(acc.at[rows], out_hbm.at[rows])
            plsc.subcore_barrier()

        for rep in range(nrep):
            @pl.when(c == 0)
            def _():
                run(ys[2 * rep], outs[2 * rep])

            @pl.when(c == 1)
            def _():
                run(ys[2 * rep + 1], outs[2 * rep + 1])

    return pl.kernel(
        body,
        out_type=[jax.ShapeDtypeStruct((n, d), F32)] * (2 * nrep),
        mesh=_sc_mesh(),
        scratch_types=[
            pltpu.VMEM((nb, _BB), jnp.int32),
            pltpu.VMEM((nb, _BB), jnp.int32),
            pltpu.VMEM((nb, _BB), F32),
            pltpu.VMEM((_BB, d), F32),
            pltpu.VMEM((_BB, d), F32),
            pltpu.VMEM_SHARED((n, d), F32),
            pltpu.VMEM_SHARED((n, d), F32),
            pltpu.SemaphoreType.DMA,
            pltpu.SemaphoreType.DMA,
            pltpu.SemaphoreType.DMA,
            pltpu.SemaphoreType.DMA,
        ],
        compiler_params=pltpu.CompilerParams(use_tc_tiling_on_sc=False,
                                             needs_layout_passes=False),
        name=f"sc_edge_agg_{d}",
    )


def _t1_call(x, W1, degp):
    n, f = x.shape
    h = W1.shape[1]
    rb = 1000

    hq = h // 4

    def body(x_ref, w_ref, degp_ref, dinv_ref, y0_ref, y1_ref, y2_ref, y3_ref):
        deg = 1.0 + degp_ref[:, 0:1] + degp_ref[:, 1:2]
        dinv = lax.rsqrt(deg)
        dinv_ref[...] = dinv
        y = dinv * jnp.dot(x_ref[...], w_ref[...], preferred_element_type=F32)
        y0_ref[...] = y[:, 0 * hq:1 * hq]
        y1_ref[...] = y[:, 1 * hq:2 * hq]
        y2_ref[...] = y[:, 2 * hq:3 * hq]
        y3_ref[...] = y[:, 3 * hq:4 * hq]

    return pl.pallas_call(
        body,
        grid=(n // rb,),
        in_specs=[pl.BlockSpec((rb, f), lambda i: (i, 0)),
                  pl.BlockSpec((f, h), lambda i: (0, 0)),
                  pl.BlockSpec((rb, 2), lambda i: (i, 0))],
        out_specs=[pl.BlockSpec((rb, 1), lambda i: (i, 0))]
        + [pl.BlockSpec((rb, hq), lambda i: (i, 0))] * 4,
        out_shape=[jax.ShapeDtypeStruct((n, 1), F32)]
        + [jax.ShapeDtypeStruct((n, hq), F32)] * 4,
    )(x, W1, degp)


def _t2_call(aggs, dinv, b1r, Wc):
    n, hq = aggs[0].shape
    h = 4 * hq
    d2 = Wc.shape[1]
    rb = 1000

    def body(a0_ref, a1_ref, a2_ref, a3_ref, dinv_ref, b1_ref, wc_ref,
             ya_ref, yb_ref):
        dinv = dinv_ref[...]
        y2 = jnp.zeros((rb, d2), F32)
        for k, a_ref in enumerate((a0_ref, a1_ref, a2_ref, a3_ref)):
            x1k = _tc_sigmoid(dinv * a_ref[...]
                              + b1_ref[:, k * hq:(k + 1) * hq])
            y2 = y2 + jnp.dot(x1k, wc_ref[k * hq:(k + 1) * hq, :],
                              preferred_element_type=F32)
        y2 = dinv * y2
        ya_ref[...] = y2[:, : d2 // 2]
        yb_ref[...] = y2[:, d2 // 2:]

    return pl.pallas_call(
        body,
        grid=(n // rb,),
        in_specs=[pl.BlockSpec((rb, hq), lambda i: (i, 0))] * 4
        + [pl.BlockSpec((rb, 1), lambda i: (i, 0)),
           pl.BlockSpec((1, h), lambda i: (0, 0)),
           pl.BlockSpec((h, d2), lambda i: (0, 0))],
        out_specs=[pl.BlockSpec((rb, d2 // 2), lambda i: (i, 0)),
                   pl.BlockSpec((rb, d2 // 2), lambda i: (i, 0))],
        out_shape=[jax.ShapeDtypeStruct((n, d2 // 2), F32),
                   jax.ShapeDtypeStruct((n, d2 // 2), F32)],
    )(*aggs, dinv, b1r, Wc)


def _t3_call(agg2a, agg2b, dinv, b2r, b3r):
    n, o = agg2a.shape
    rb = 200

    def body(ai_ref, afull_ref, bi_ref, dvi_ref, dvfull_ref, b2_ref, b3_ref,
             adj_ref, mu_ref, lv_ref):
        mu_i = _tc_sigmoid(dvi_ref[...] * ai_ref[...] + b2_ref[...])
        lv_i = _tc_sigmoid(dvi_ref[...] * bi_ref[...] + b3_ref[...])
        mu_all = _tc_sigmoid(dvfull_ref[...] * afull_ref[...] + b2_ref[...])
        mu_ref[...] = mu_i
        lv_ref[...] = lv_i
        prod = lax.dot_general(mu_i, mu_all, (((1,), (1,)), ((), ())),
                               preferred_element_type=F32)
        adj_ref[...] = _tc_sigmoid(prod)

    return pl.pallas_call(
        body,
        grid=(n // rb,),
        in_specs=[pl.BlockSpec((rb, o), lambda i: (i, 0)),
                  pl.BlockSpec((n, o), lambda i: (0, 0)),
                  pl.BlockSpec((rb, o), lambda i: (i, 0)),
                  pl.BlockSpec((rb, 1), lambda i: (i, 0)),
                  pl.BlockSpec((n, 1), lambda i: (0, 0)),
                  pl.BlockSpec((1, o), lambda i: (0, 0)),
                  pl.BlockSpec((1, o), lambda i: (0, 0))],
        out_specs=[pl.BlockSpec((rb, n), lambda i: (i, 0)),
                   pl.BlockSpec((rb, o), lambda i: (i, 0)),
                   pl.BlockSpec((rb, o), lambda i: (i, 0))],
        out_shape=[jax.ShapeDtypeStruct((n, n), F32),
                   jax.ShapeDtypeStruct((n, o), F32),
                   jax.ShapeDtypeStruct((n, o), F32)],
    )(agg2a, agg2a, agg2b, dinv, dinv, b2r, b3r)


def kernel(x, edge_index, edge_weight, W1, b1, W2, b2, W3, b3):
    n, f = x.shape
    h = W1.shape[1]
    o = W2.shape[1]
    e = edge_weight.shape[0]

    row = edge_index[0].astype(jnp.int32)
    col = edge_index[1].astype(jnp.int32)
    ew = edge_weight.astype(F32)

    nb = -(-(-(-e // (_NS * _BB))) // _NBUF) * _NBUF
    ep = _NS * nb * _BB
    chunk = -(-e // (_NC * _NS * _LN)) * _LN
    epd = _NC * _NS * chunk
    epmax = max(ep, epd)
    padlen = epmax - e
    rowp = jnp.concatenate([row, jnp.zeros((padlen,), jnp.int32)])
    colp = jnp.concatenate([col, jnp.zeros((padlen,), jnp.int32)])
    ewp = jnp.concatenate([ew, jnp.zeros((padlen,), F32)])
    row_s = rowp[:ep].reshape(_NS, nb, _BB)
    col_s = colp[:ep].reshape(_NS, nb, _BB)
    ew_s = ewp[:ep].reshape(_NS, nb, _BB)
    col_d = colp[:epd].reshape(_NC * _NS, chunk)
    ew_d = ewp[:epd].reshape(_NC * _NS, chunk)

    npad = -(-n // (_NS * _LN)) * (_NS * _LN)

    degp = _make_deg_kernel(npad, epd)(col_d, ew_d)
    degp2 = degp[:, :n].T

    dinv, y1q0, y1q1, y1q2, y1q3 = _t1_call(x, W1, degp2)
    agg1 = _make_agg_kernel(n, h // 4, nb, 2)(
        y1q0, y1q1, y1q2, y1q3, row_s, col_s, ew_s)

    Wc = jnp.concatenate([W2, W3], axis=1)
    y2a, y2b = _t2_call(agg1, dinv, b1.reshape(1, h), Wc)
    agg2a, agg2b = _make_agg_kernel(n, o, nb, 1)(
        y2a, y2b, row_s, col_s, ew_s)

    adj, mu, logvar = _t3_call(agg2a, agg2b, dinv,
                               b2.reshape(1, o), b3.reshape(1, o))
    return mu, logvar, mu, adj


# speedup vs baseline: 1.1071x; 1.0699x over previous
import functools

import jax
import jax.numpy as jnp
from jax import lax
from jax.experimental import pallas as pl
from jax.experimental.pallas import tpu as pltpu
from jax.experimental.pallas import tpu_sc as plsc

F32 = jnp.float32
_NC = 2
_NS = 16
_LN = 16
_BB = 128


def _tc_sigmoid(v):
    return 0.5 * jnp.tanh(0.5 * v) + 0.5


def _lane_bcast(v, lane):
    idx = jnp.full((_LN, 1), lane, jnp.int32)
    dn = lax.GatherDimensionNumbers(offset_dims=(), collapsed_slice_dims=(0,),
                                    start_index_map=(0,))
    return lax.gather(v, idx, dn, (1,),
                      mode=lax.GatherScatterMode.PROMISE_IN_BOUNDS)


def _sc_mesh():
    return plsc.VectorSubcoreMesh(core_axis_name="c", subcore_axis_name="s",
                                  num_cores=_NC, num_subcores=_NS)


@functools.lru_cache(maxsize=None)
def _make_deg_kernel(npad, epd):
    chunk = epd // (_NC * _NS)
    nvec = chunk // _LN
    per_tile = npad // _NS

    def body(col_hbm, ew_hbm, out_hbm, colv, eww, acc, redbuf, shared):
        c = lax.axis_index("c")
        s = lax.axis_index("s")
        wid = c * _NS + s

        def zero(i, _):
            acc[pl.ds(i * _LN, _LN)] = jnp.zeros((_LN,), F32)
            return 0
        lax.fori_loop(0, npad // _LN, zero, 0)

        pltpu.sync_copy(col_hbm.at[wid], colv)
        pltpu.sync_copy(ew_hbm.at[wid], eww)

        def step(i, _):
            idx = colv[pl.ds(i * _LN, _LN)]
            w = eww[pl.ds(i * _LN, _LN)]
            plsc.addupdate_scatter(acc, [idx], w)
            return 0
        lax.fori_loop(0, nvec, step, 0)

        pltpu.sync_copy(acc, shared.at[s])
        plsc.subcore_barrier()

        base = s * per_tile
        for t in range(_NS):
            pltpu.sync_copy(shared.at[t, pl.ds(base, per_tile)], redbuf.at[t])

        def red(j, _):
            sl = pl.ds(j * _LN, _LN)
            v = redbuf[0, sl]
            for t in range(1, _NS):
                v = v + redbuf[t, sl]
            acc[sl] = v
            return 0
        lax.fori_loop(0, per_tile // _LN, red, 0)
        pltpu.sync_copy(acc.at[pl.ds(0, per_tile)],
                        out_hbm.at[c, pl.ds(base, per_tile)])

    return pl.kernel(
        body,
        out_type=jax.ShapeDtypeStruct((_NC, npad), F32),
        mesh=_sc_mesh(),
        scratch_types=[
            pltpu.VMEM((chunk,), jnp.int32),
            pltpu.VMEM((chunk,), F32),
            pltpu.VMEM((npad,), F32),
            pltpu.VMEM((_NS, per_tile), F32),
            pltpu.VMEM_SHARED((_NS, npad), F32),
        ],
        compiler_params=pltpu.CompilerParams(needs_layout_passes=False),
        name="sc_degree",
    )


@functools.lru_cache(maxsize=None)
def _make_agg_kernel(n, d, nb):
    rows_per_tile = n // _NS
    nvec = d // _LN

    def body(ya_hbm, yb_hbm, row_hbm, col_hbm, ew_hbm, outa_hbm, outb_hbm,
             idxr, idxc, eww, rbuf, acc, sem):
        c = lax.axis_index("c")
        s = lax.axis_index("s")
        base = s * rows_per_tile
        rows = pl.ds(base, rows_per_tile)

        pltpu.sync_copy(row_hbm.at[s], idxr)
        pltpu.sync_copy(col_hbm.at[s], idxc)
        pltpu.sync_copy(ew_hbm.at[s], eww)

        def run(y_hbm, out_hbm):
            pltpu.sync_copy(y_hbm.at[rows], acc.at[rows])
            plsc.subcore_barrier()

            def blk(j, _):
                pltpu.async_copy(y_hbm.at[idxr.at[j]], rbuf, sem).wait()

                def grp(g, _):
                    wv = eww[j, pl.ds(g * _LN, _LN)]
                    for l in range(_LN):
                        w = _lane_bcast(wv, l)
                        eb = g * _LN + l
                        for f in range(nvec):
                            sl = pl.ds(f * _LN, _LN)
                            rbuf[eb, sl] = rbuf[eb, sl] * w
                    return 0
                lax.fori_loop(0, _BB // _LN, grp, 0)

                pltpu.sync_copy(rbuf, acc.at[idxc.at[j]], add=True)
                return 0
            lax.fori_loop(0, nb, blk, 0)

            plsc.subcore_barrier()
            pltpu.sync_copy(acc.at[rows], out_hbm.at[rows])

        @pl.when(c == 0)
        def _():
            run(ya_hbm, outa_hbm)

        @pl.when(c == 1)
        def _():
            run(yb_hbm, outb_hbm)

    return pl.kernel(
        body,
        out_type=[jax.ShapeDtypeStruct((n, d), F32),
                  jax.ShapeDtypeStruct((n, d), F32)],
        mesh=_sc_mesh(),
        scratch_types=[
            pltpu.VMEM((nb, _BB), jnp.int32),
            pltpu.VMEM((nb, _BB), jnp.int32),
            pltpu.VMEM((nb, _BB), F32),
            pltpu.VMEM((_BB, d), F32),
            pltpu.VMEM_SHARED((n, d), F32),
            pltpu.SemaphoreType.DMA,
        ],
        compiler_params=pltpu.CompilerParams(use_tc_tiling_on_sc=False,
                                             needs_layout_passes=False),
        name=f"sc_edge_agg_{d}",
    )


def _t1_call(x, W1, degp):
    n, f = x.shape
    h = W1.shape[1]
    rb = 1000
    hh = h // 2

    def body(x_ref, w_ref, degp_ref, dinv_ref, ya_ref, yb_ref):
        deg = 1.0 + degp_ref[:, 0:1] + degp_ref[:, 1:2]
        dinv = lax.rsqrt(deg)
        dinv_ref[...] = dinv
        y = dinv * jnp.dot(x_ref[...], w_ref[...], preferred_element_type=F32)
        ya_ref[...] = y[:, :hh]
        yb_ref[...] = y[:, hh:]

    return pl.pallas_call(
        body,
        grid=(n // rb,),
        in_specs=[pl.BlockSpec((rb, f), lambda i: (i, 0)),
                  pl.BlockSpec((f, h), lambda i: (0, 0)),
                  pl.BlockSpec((rb, 2), lambda i: (i, 0))],
        out_specs=[pl.BlockSpec((rb, 1), lambda i: (i, 0))]
        + [pl.BlockSpec((rb, hh), lambda i: (i, 0))] * 2,
        out_shape=[jax.ShapeDtypeStruct((n, 1), F32)]
        + [jax.ShapeDtypeStruct((n, hh), F32)] * 2,
    )(x, W1, degp)


def _t2_call(agg1a, agg1b, dinv, b1r, Wc):
    n, hh = agg1a.shape
    h = 2 * hh
    d2 = Wc.shape[1]
    rb = 1000

    def body(aa_ref, ab_ref, dinv_ref, b1_ref, wc_ref, ya_ref, yb_ref):
        dinv = dinv_ref[...]
        x1a = _tc_sigmoid(dinv * aa_ref[...] + b1_ref[:, :hh])
        x1b = _tc_sigmoid(dinv * ab_ref[...] + b1_ref[:, hh:])
        y2 = jnp.dot(x1a, wc_ref[:hh, :], preferred_element_type=F32)
        y2 = y2 + jnp.dot(x1b, wc_ref[hh:, :], preferred_element_type=F32)
        y2 = dinv * y2
        ya_ref[...] = y2[:, : d2 // 2]
        yb_ref[...] = y2[:, d2 // 2:]

    return pl.pallas_call(
        body,
        grid=(n // rb,),
        in_specs=[pl.BlockSpec((rb, hh), lambda i: (i, 0)),
                  pl.BlockSpec((rb, hh), lambda i: (i, 0)),
                  pl.BlockSpec((rb, 1), lambda i: (i, 0)),
                  pl.BlockSpec((1, h), lambda i: (0, 0)),
                  pl.BlockSpec((h, d2), lambda i: (0, 0))],
        out_specs=[pl.BlockSpec((rb, d2 // 2), lambda i: (i, 0)),
                   pl.BlockSpec((rb, d2 // 2), lambda i: (i, 0))],
        out_shape=[jax.ShapeDtypeStruct((n, d2 // 2), F32),
                   jax.ShapeDtypeStruct((n, d2 // 2), F32)],
    )(agg1a, agg1b, dinv, b1r, Wc)


def _t3a_call(agg2a, agg2b, dinv, b2r, b3r):
    n, o = agg2a.shape
    rb = 1000

    def body(a_ref, b_ref, dv_ref, b2_ref, b3_ref, mu_ref, lv_ref, muh_ref):
        dv = dv_ref[...]
        mu = _tc_sigmoid(dv * a_ref[...] + b2_ref[...])
        mu_ref[...] = mu
        lv_ref[...] = _tc_sigmoid(dv * b_ref[...] + b3_ref[...])
        muh_ref[...] = 0.5 * mu

    return pl.pallas_call(
        body,
        grid=(n // rb,),
        in_specs=[pl.BlockSpec((rb, o), lambda i: (i, 0)),
                  pl.BlockSpec((rb, o), lambda i: (i, 0)),
                  pl.BlockSpec((rb, 1), lambda i: (i, 0)),
                  pl.BlockSpec((1, o), lambda i: (0, 0)),
                  pl.BlockSpec((1, o), lambda i: (0, 0))],
        out_specs=[pl.BlockSpec((rb, o), lambda i: (i, 0))] * 3,
        out_shape=[jax.ShapeDtypeStruct((n, o), F32)] * 3,
    )(agg2a, agg2b, dinv, b2r, b3r)


def _t3b_call(muh, mu):
    n, o = mu.shape
    rb = 200

    def body(mh_ref, mall_ref, adj_ref):
        prod = lax.dot_general(mh_ref[...], mall_ref[...],
                               (((1,), (1,)), ((), ())),
                               preferred_element_type=F32)
        adj_ref[...] = 0.5 * jnp.tanh(prod) + 0.5

    return pl.pallas_call(
        body,
        grid=(n // rb,),
        in_specs=[pl.BlockSpec((rb, o), lambda i: (i, 0)),
                  pl.BlockSpec((n, o), lambda i: (0, 0))],
        out_specs=pl.BlockSpec((rb, n), lambda i: (i, 0)),
        out_shape=jax.ShapeDtypeStruct((n, n), F32),
    )(muh, mu)


def kernel(x, edge_index, edge_weight, W1, b1, W2, b2, W3, b3):
    n, f = x.shape
    h = W1.shape[1]
    o = W2.shape[1]
    e = edge_weight.shape[0]

    row = edge_index[0].astype(jnp.int32)
    col = edge_index[1].astype(jnp.int32)
    ew = edge_weight.astype(F32)

    nb = -(-e // (_NS * _BB))
    ep = _NS * nb * _BB
    chunk = -(-e // (_NC * _NS * _LN)) * _LN
    epd = _NC * _NS * chunk
    epmax = max(ep, epd)
    padlen = epmax - e
    rowp = jnp.concatenate([row, jnp.zeros((padlen,), jnp.int32)])
    colp = jnp.concatenate([col, jnp.zeros((padlen,), jnp.int32)])
    ewp = jnp.concatenate([ew, jnp.zeros((padlen,), F32)])
    row_s = rowp[:ep].reshape(_NS, nb, _BB)
    col_s = colp[:ep].reshape(_NS, nb, _BB)
    ew_s = ewp[:ep].reshape(_NS, nb, _BB)
    col_d = colp[:epd].reshape(_NC * _NS, chunk)
    ew_d = ewp[:epd].reshape(_NC * _NS, chunk)

    npad = -(-n // (_NS * _LN)) * (_NS * _LN)

    degp = _make_deg_kernel(npad, epd)(col_d, ew_d)
    degp2 = degp[:, :n].T

    dinv, y1a, y1b = _t1_call(x, W1, degp2)
    agg1a, agg1b = _make_agg_kernel(n, h // 2, nb)(
        y1a, y1b, row_s, col_s, ew_s)

    Wc = jnp.concatenate([W2, W3], axis=1)
    y2a, y2b = _t2_call(agg1a, agg1b, dinv, b1.reshape(1, h), Wc)
    agg2a, agg2b = _make_agg_kernel(n, o, nb)(
        y2a, y2b, row_s, col_s, ew_s)

    mu, logvar, muh = _t3a_call(agg2a, agg2b, dinv,
                                b2.reshape(1, o), b3.reshape(1, o))
    adj = _t3b_call(muh, mu)
    return mu, logvar, mu, adj
